# baseline (speedup 1.0000x reference)
import functools

import jax
import jax.numpy as jnp
from jax import lax
from jax.experimental import pallas as pl
from jax.experimental.pallas import tpu as pltpu
from jax.experimental.pallas import tpu_sc as plsc

B = 16384
D = 64
NC = 2
NS = 16
NW = NC * NS
BPW = B // NW
CHUNK = 128
NCHUNK = BPW // CHUNK
LANES = 16

_MESH = plsc.VectorSubcoreMesh(core_axis_name="c", subcore_axis_name="s")


@functools.partial(
    pl.kernel,
    mesh=_MESH,
    out_type=jax.ShapeDtypeStruct((B, 6 * D), jnp.float32),
    scratch_types=[
        pltpu.VMEM((6, BPW), jnp.int32),
        pltpu.VMEM((NCHUNK, CHUNK), jnp.int32),
        pltpu.VMEM((NCHUNK, CHUNK), jnp.int32),
        pltpu.VMEM((NCHUNK, CHUNK), jnp.int32),
        pltpu.VMEM((CHUNK, 2 * D), jnp.float32),
        pltpu.VMEM((CHUNK, 2 * D), jnp.float32),
        pltpu.VMEM((CHUNK, 2 * D), jnp.float32),
        pltpu.VMEM((CHUNK, 2 * D), jnp.float32),
        pltpu.VMEM((CHUNK, 2 * D), jnp.float32),
        pltpu.VMEM((CHUNK, 2 * D), jnp.float32),
        pltpu.SemaphoreType.DMA,
        pltpu.SemaphoreType.DMA,
        pltpu.SemaphoreType.DMA,
        pltpu.SemaphoreType.DMA,
    ],
)
def _sc_embed(t12, t34, t56, c1, c2, c3, c4, c5, c6, out,
              codes, idx12, idx34, idx56,
              b12a, b34a, b56a, b12b, b34b, b56b,
              sga, sgb, swa, swb):
    wid = lax.axis_index("s") * NC + lax.axis_index("c")
    base = wid * BPW

    pltpu.sync_copy(c1.at[pl.ds(base, BPW)], codes.at[0])
    pltpu.sync_copy(c2.at[pl.ds(base, BPW)], codes.at[1])
    pltpu.sync_copy(c3.at[pl.ds(base, BPW)], codes.at[2])
    pltpu.sync_copy(c4.at[pl.ds(base, BPW)], codes.at[3])
    pltpu.sync_copy(c5.at[pl.ds(base, BPW)], codes.at[4])
    pltpu.sync_copy(c6.at[pl.ds(base, BPW)], codes.at[5])

    for c in range(NCHUNK):
        for k in range(CHUNK // LANES):
            s = c * CHUNK + k * LANES
            sl = pl.ds(s, LANES)
            ksl = pl.ds(k * LANES, LANES)
            idx12[c, ksl] = codes[0, sl] * 11 + codes[1, sl]
            idx34[c, ksl] = codes[2, sl] * 12 + codes[3, sl]
            idx56[c, ksl] = codes[4, sl] * 24 + codes[5, sl]

    tabs = (t12, t34, t56)
    idxs = (idx12, idx34, idx56)
    bufs = ((b12a, b34a, b56a), (b12b, b34b, b56b))
    gsem = (sga, sgb)
    wsem = (swa, swb)

    def issue_gathers(c, s):
        return [pltpu.async_copy(tabs[p].at[idxs[p].at[c]], bufs[s][p], gsem[s])
                for p in range(3)]

    def issue_writes(c, s):
        r0 = base + c * CHUNK
        return [pltpu.async_copy(
                    bufs[s][p],
                    out.at[pl.ds(r0, CHUNK), pl.ds(p * 2 * D, 2 * D)],
                    wsem[s])
                for p in range(3)]

    pend_g = [None, None]
    pend_w = [None, None]
    pend_g[0] = issue_gathers(0, 0)
    for c in range(NCHUNK):
        cur = c & 1
        nxt = 1 - cur
        if c + 1 < NCHUNK:
            if pend_w[nxt] is not None:
                for w in pend_w[nxt]:
                    w.wait()
                pend_w[nxt] = None
            pend_g[nxt] = issue_gathers(c + 1, nxt)
        for g in pend_g[cur]:
            g.wait()
        pend_w[cur] = issue_writes(c, cur)
    for s in range(2):
        if pend_w[s] is not None:
            for w in pend_w[s]:
                w.wait()


def kernel(code_holiday, code_weather, code_weather_detail, code_month,
           code_dayofweek, code_hour, W_holiday, W_weather, W_weather_detail,
           W_month, W_dayofweek, W_hour):
    t12 = jnp.concatenate([
        jnp.broadcast_to(W_holiday[:, None, :], (12, 11, D)),
        jnp.broadcast_to(W_weather[None, :, :], (12, 11, D)),
    ], axis=2).reshape(12 * 11, 2 * D)
    t34 = jnp.concatenate([
        jnp.broadcast_to(W_weather_detail[:, None, :], (38, 12, D)),
        jnp.broadcast_to(W_month[None, :, :], (38, 12, D)),
    ], axis=2).reshape(38 * 12, 2 * D)
    t56 = jnp.concatenate([
        jnp.broadcast_to(W_dayofweek[:, None, :], (7, 24, D)),
        jnp.broadcast_to(W_hour[None, :, :], (7, 24, D)),
    ], axis=2).reshape(7 * 24, 2 * D)

    codes = [c.astype(jnp.int32) for c in (
        code_holiday, code_weather, code_weather_detail,
        code_month, code_dayofweek, code_hour)]
    return _sc_embed(t12, t34, t56, *codes)
